# Initial kernel scaffold; baseline (speedup 1.0000x reference)
import jax
import jax.numpy as jnp
from jax.experimental import pallas as pl


def kernel(user_emb, item_emb, adj_indices, adj_values):
    raise NotImplementedError("write your pallas kernel here")


# speedup vs baseline: 1.9832x; 1.9832x over previous
import functools

import jax
import jax.numpy as jnp
from jax import lax
from jax.experimental import pallas as pl
from jax.experimental.pallas import tpu as pltpu
from jax.experimental.pallas import tpu_sc as plsc

NUM_USERS = 20000
NUM_ITEMS = 30000
N_NODES = NUM_USERS + NUM_ITEMS
D = 64
N_LAYERS = 3

NC = 2
NS = 16
L = 16

HALF = 25000
HP = 25088
NPAD = 2 * HP
ROWS_PER_TILE = HP // NS
ACC_ROWS = HP + 128

K = 128
NCH = 391
EDGES_PER_TILE = K * NCH
EPAD = NS * EDGES_PER_TILE

_mesh = plsc.VectorSubcoreMesh(core_axis_name="c", subcore_axis_name="s")

_GDN = lax.GatherDimensionNumbers(
    offset_dims=(), collapsed_slice_dims=(0,), start_index_map=(0,))


def _lane_bcast(vec, lane):
    idx = jnp.full((L, 1), lane, jnp.int32)
    return lax.gather(vec, idx, dimension_numbers=_GDN, slice_sizes=(1,),
                      mode=lax.GatherScatterMode.PROMISE_IN_BOUNDS)


def _layer_body(ego, srcp, dstp, val, out,
                acc, rows_v, srcp_v, dstp_v, lidx_v, val_v, sem):
    c = lax.axis_index("c")
    s = lax.axis_index("s")

    def zero_row(r, _):
        for cc in range(4):
            rows_v[r, pl.ds(cc * L, L)] = jnp.zeros((L,), jnp.float32)
        return 0
    lax.fori_loop(0, K, zero_row, 0)
    zslice = ACC_ROWS // NS
    zbase = s * zslice
    for j in range(zslice // K):
        pltpu.sync_copy(rows_v, acc.at[pl.ds(zbase + j * K, K)])
    ztail = zslice - (zslice // K) * K
    pltpu.sync_copy(rows_v.at[pl.ds(0, ztail)],
                    acc.at[pl.ds(zbase + (zslice // K) * K, ztail)])
    plsc.subcore_barrier()

    base = s * EDGES_PER_TILE
    my_lo = c * HP
    dummy_base = jnp.int32(HP) + lax.iota(jnp.int32, L)

    def chunk(i, _):
        e0 = base + i * K
        pltpu.sync_copy(srcp.at[pl.ds(e0, K)], srcp_v)
        pltpu.sync_copy(dstp.at[pl.ds(e0, K)], dstp_v)
        pltpu.sync_copy(val.at[pl.ds(e0, K)], val_v)
        pltpu.async_copy(ego.at[srcp_v], rows_v, sem).wait()
        for j in range(K // L):
            d = dstp_v[pl.ds(j * L, L)] - my_lo
            ok = (d >= 0) & (d < HP)
            lidx_v[pl.ds(j * L, L)] = jnp.where(ok, d, dummy_base)
        def scale(g, _):
            vv = val_v[pl.ds(g * L, L)]
            for k16 in range(L):
                v = _lane_bcast(vv, k16)
                k = g * L + k16
                for cc in range(4):
                    rows_v[k, pl.ds(cc * L, L)] = rows_v[k, pl.ds(cc * L, L)] * v
            return 0
        lax.fori_loop(0, K // L, scale, 0)
        pltpu.sync_copy(rows_v, acc.at[lidx_v], add=True)
        return 0

    lax.fori_loop(0, NCH, chunk, 0)
    plsc.subcore_barrier()

    r0 = s * ROWS_PER_TILE
    pltpu.sync_copy(acc.at[pl.ds(r0, ROWS_PER_TILE)],
                    out.at[pl.ds(c * HP + r0, ROWS_PER_TILE)])


_layer = pl.kernel(
    _layer_body,
    mesh=_mesh,
    compiler_params=pltpu.CompilerParams(use_tc_tiling_on_sc=False),
    out_type=jax.ShapeDtypeStruct((NPAD, D), jnp.float32),
    scratch_types=[
        pltpu.VMEM_SHARED((ACC_ROWS, D), jnp.float32),
        pltpu.VMEM((K, D), jnp.float32),
        pltpu.VMEM((K,), jnp.int32),
        pltpu.VMEM((K,), jnp.int32),
        pltpu.VMEM((K,), jnp.int32),
        pltpu.VMEM((K,), jnp.float32),
        pltpu.SemaphoreType.DMA,
    ],
)


MROWS = NPAD // (NC * NS)
MK = 128


def _mean_body(e0, e1, e2, e3, out, b0, b1, b2, b3):
    c = lax.axis_index("c")
    s = lax.axis_index("s")
    wid = s * NC + c
    r0 = wid * MROWS

    def do_chunk(off, nrows):
        for b, e in ((b0, e0), (b1, e1), (b2, e2), (b3, e3)):
            pltpu.sync_copy(e.at[pl.ds(r0 + off, nrows)], b.at[pl.ds(0, nrows)])

        def srow(r, _):
            for cc in range(4):
                sl = pl.ds(cc * L, L)
                acc = ((b0[r, sl] + b1[r, sl]) + (b2[r, sl] + b3[r, sl]))
                b0[r, sl] = acc * 0.25
            return 0
        lax.fori_loop(0, nrows, srow, 0)
        pltpu.sync_copy(b0.at[pl.ds(0, nrows)], out.at[pl.ds(r0 + off, nrows)])

    def mchunk(i, _):
        do_chunk(i * MK, MK)
        return 0
    lax.fori_loop(0, MROWS // MK, mchunk, 0)
    do_chunk((MROWS // MK) * MK, MROWS - (MROWS // MK) * MK)


_mean = pl.kernel(
    _mean_body,
    mesh=_mesh,
    out_type=jax.ShapeDtypeStruct((NPAD, D), jnp.float32),
    scratch_types=[
        pltpu.VMEM((MK, D), jnp.float32),
        pltpu.VMEM((MK, D), jnp.float32),
        pltpu.VMEM((MK, D), jnp.float32),
        pltpu.VMEM((MK, D), jnp.float32),
    ],
)


@functools.partial(jax.jit, static_argnums=())
def kernel(user_emb, item_emb, adj_indices, adj_values):
    ego0 = jnp.concatenate([user_emb, item_emb], axis=0)
    ego0p = jnp.concatenate([
        ego0[:HALF], jnp.zeros((HP - HALF, D), jnp.float32),
        ego0[HALF:], jnp.zeros((HP - HALF, D), jnp.float32),
    ], axis=0)
    src = adj_indices[0]
    dst = adj_indices[1]
    srcp = src + (HP - HALF) * (src >= HALF).astype(jnp.int32)
    dstp = dst + (HP - HALF) * (dst >= HALF).astype(jnp.int32)
    e = src.shape[0]
    pad = EPAD - e
    srcp = jnp.concatenate([srcp, jnp.zeros((pad,), jnp.int32)])
    dstp = jnp.concatenate([dstp, jnp.full((pad,), 1 << 28, jnp.int32)])
    valp = jnp.concatenate([adj_values, jnp.zeros((pad,), jnp.float32)])

    ego1 = _layer(ego0p, srcp, dstp, valp)
    ego2 = _layer(ego1, srcp, dstp, valp)
    ego3 = _layer(ego2, srcp, dstp, valp)
    finalp = _mean(ego0p, ego1, ego2, ego3)

    final_user = finalp[:NUM_USERS]
    final_item = jnp.concatenate(
        [finalp[NUM_USERS:HALF], finalp[HP:HP + NUM_ITEMS - (HALF - NUM_USERS)]],
        axis=0)
    return (final_user, final_item)


# speedup vs baseline: 4.5072x; 2.2727x over previous
import jax
import jax.numpy as jnp
from jax import lax
from jax.experimental import pallas as pl
from jax.experimental.pallas import tpu as pltpu
from jax.experimental.pallas import tpu_sc as plsc

NUM_USERS = 20000
NUM_ITEMS = 30000
D = 64
DH = 32
L = 16
NS = 16

HALF = 25000
HP = 25088
NPAD = 2 * HP
RPT = NPAD // NS

K = 128
G = 2
SK = G * K
NSC = 196
EDGES_PER_TILE = SK * NSC
EPAD = NS * EDGES_PER_TILE

MR = 64

_mesh = plsc.VectorSubcoreMesh(core_axis_name="c", subcore_axis_name="s")

_GDN = lax.GatherDimensionNumbers(
    offset_dims=(), collapsed_slice_dims=(0,), start_index_map=(0,))


def _lane_bcast(vec, lane):
    idx = jnp.full((L, 1), lane, jnp.int32)
    return lax.gather(vec, idx, dimension_numbers=_GDN, slice_sizes=(1,),
                      mode=lax.GatherScatterMode.PROMISE_IN_BOUNDS)


def _zero_acc(acc, rows2, s):
    def zero_row(r, _):
        rows2[0, r, pl.ds(0, L)] = jnp.zeros((L,), jnp.float32)
        rows2[0, r, pl.ds(L, L)] = jnp.zeros((L,), jnp.float32)
        return 0
    lax.fori_loop(0, SK, zero_row, 0)
    zbase = s * RPT
    for j in range(RPT // SK):
        pltpu.sync_copy(rows2.at[0], acc.at[pl.ds(zbase + j * SK, SK)])
    ztail = RPT - (RPT // SK) * SK
    pltpu.sync_copy(rows2.at[0, pl.ds(0, ztail)],
                    acc.at[pl.ds(zbase + (RPT // SK) * SK, ztail)])


def _edge_scan(ego, srcp, dstp, val, acc,
               rows2, srcp2, dstp2, val2, gsem0, gsem1, s):
    base = s * EDGES_PER_TILE
    gsems = (gsem0, gsem1)

    def load_idx(j, p):
        e0 = base + j * SK
        for g in range(G):
            pltpu.sync_copy(srcp.at[pl.ds(e0 + g * K, K)], srcp2.at[p, g])
            pltpu.sync_copy(dstp.at[pl.ds(e0 + g * K, K)], dstp2.at[p, g])
            pltpu.sync_copy(val.at[pl.ds(e0 + g * K, K)], val2.at[p, g])

    def issue_gathers(p):
        for g in range(G):
            pltpu.async_copy(ego.at[srcp2.at[p, g]],
                             rows2.at[p, pl.ds(g * K, K)], gsems[p])

    def wait_gathers(p):
        for g in range(G):
            pltpu.make_async_copy(ego.at[pl.ds(0, K)],
                                  rows2.at[p, pl.ds(g * K, K)],
                                  gsems[p]).wait()

    def scale(p):
        for g in range(G):
            def sgroup(gr, _, g=g):
                vv = val2[p, g, pl.ds(gr * L, L)]
                for k16 in range(L):
                    v = _lane_bcast(vv, k16)
                    k = g * K + gr * L + k16
                    rows2[p, k, pl.ds(0, L)] = rows2[p, k, pl.ds(0, L)] * v
                    rows2[p, k, pl.ds(L, L)] = rows2[p, k, pl.ds(L, L)] * v
                return 0
            lax.fori_loop(0, K // L, sgroup, 0)

    def scatter(p):
        for g in range(G):
            pltpu.sync_copy(rows2.at[p, pl.ds(g * K, K)],
                            acc.at[dstp2.at[p, g]], add=True)

    load_idx(0, 0)
    issue_gathers(0)

    def super_pair(jj, _):
        for p in (0, 1):
            j = 2 * jj + p
            jn = jnp.where(j + 1 >= NSC, 0, j + 1)
            load_idx(jn, 1 - p)
            issue_gathers(1 - p)
            wait_gathers(p)
            scale(p)
            scatter(p)
        return 0
    lax.fori_loop(0, NSC // 2, super_pair, 0)
    wait_gathers(0)


def _layer_body(ego_lo, ego_hi, srcp, dstp, val, out_lo, out_hi,
                acc, rows2, srcp2, dstp2, val2, gsem0, gsem1):
    c = lax.axis_index("c")
    s = lax.axis_index("s")
    _zero_acc(acc, rows2, s)
    plsc.subcore_barrier()

    @pl.when(c == 0)
    def _():
        _edge_scan(ego_lo, srcp, dstp, val, acc,
                   rows2, srcp2, dstp2, val2, gsem0, gsem1, s)

    @pl.when(c == 1)
    def _():
        _edge_scan(ego_hi, srcp, dstp, val, acc,
                   rows2, srcp2, dstp2, val2, gsem0, gsem1, s)
    plsc.subcore_barrier()
    r0 = s * RPT

    @pl.when(c == 0)
    def _():
        pltpu.sync_copy(acc.at[pl.ds(r0, RPT)], out_lo.at[pl.ds(r0, RPT)])

    @pl.when(c == 1)
    def _():
        pltpu.sync_copy(acc.at[pl.ds(r0, RPT)], out_hi.at[pl.ds(r0, RPT)])


_half_t = jax.ShapeDtypeStruct((NPAD, DH), jnp.float32)

_layer = pl.kernel(
    _layer_body,
    mesh=_mesh,
    compiler_params=pltpu.CompilerParams(use_tc_tiling_on_sc=False),
    out_type=[_half_t, _half_t],
    scratch_types=[
        pltpu.VMEM_SHARED((NPAD, DH), jnp.float32),
        pltpu.VMEM((2, SK, DH), jnp.float32),
        pltpu.VMEM((2, G, K), jnp.int32),
        pltpu.VMEM((2, G, K), jnp.int32),
        pltpu.VMEM((2, G, K), jnp.float32),
        pltpu.SemaphoreType.DMA,
        pltpu.SemaphoreType.DMA,
    ],
)


def _mean_out(e0, e1, e2, acc, out, b0, b1, b2, b3, s):
    r0 = s * RPT

    def mchunk(j, _):
        off = r0 + j * MR
        pltpu.sync_copy(e0.at[pl.ds(off, MR)], b0)
        pltpu.sync_copy(e1.at[pl.ds(off, MR)], b1)
        pltpu.sync_copy(e2.at[pl.ds(off, MR)], b2)
        pltpu.sync_copy(acc.at[pl.ds(off, MR)], b3)

        def srow(r, _):
            for cc in range(2):
                sl = pl.ds(cc * L, L)
                b0[r, sl] = ((b0[r, sl] + b1[r, sl]) +
                             (b2[r, sl] + b3[r, sl])) * 0.25
            return 0
        lax.fori_loop(0, MR, srow, 0)
        pltpu.sync_copy(b0, out.at[pl.ds(off, MR)])
        return 0

    lax.fori_loop(0, RPT // MR, mchunk, 0)


def _final_body(ego_lo, ego_hi, e0_lo, e0_hi, e1_lo, e1_hi, srcp, dstp, val,
                out_lo, out_hi,
                acc, rows2, srcp2, dstp2, val2, b0, b1, b2, b3, gsem0, gsem1):
    c = lax.axis_index("c")
    s = lax.axis_index("s")
    _zero_acc(acc, rows2, s)
    plsc.subcore_barrier()

    @pl.when(c == 0)
    def _():
        _edge_scan(ego_lo, srcp, dstp, val, acc,
                   rows2, srcp2, dstp2, val2, gsem0, gsem1, s)

    @pl.when(c == 1)
    def _():
        _edge_scan(ego_hi, srcp, dstp, val, acc,
                   rows2, srcp2, dstp2, val2, gsem0, gsem1, s)
    plsc.subcore_barrier()

    @pl.when(c == 0)
    def _():
        _mean_out(e0_lo, e1_lo, ego_lo, acc, out_lo, b0, b1, b2, b3, s)

    @pl.when(c == 1)
    def _():
        _mean_out(e0_hi, e1_hi, ego_hi, acc, out_hi, b0, b1, b2, b3, s)


_final = pl.kernel(
    _final_body,
    mesh=_mesh,
    compiler_params=pltpu.CompilerParams(use_tc_tiling_on_sc=False),
    out_type=[_half_t, _half_t],
    scratch_types=[
        pltpu.VMEM_SHARED((NPAD, DH), jnp.float32),
        pltpu.VMEM((2, SK, DH), jnp.float32),
        pltpu.VMEM((2, G, K), jnp.int32),
        pltpu.VMEM((2, G, K), jnp.int32),
        pltpu.VMEM((2, G, K), jnp.float32),
        pltpu.VMEM((MR, DH), jnp.float32),
        pltpu.VMEM((MR, DH), jnp.float32),
        pltpu.VMEM((MR, DH), jnp.float32),
        pltpu.VMEM((MR, DH), jnp.float32),
        pltpu.SemaphoreType.DMA,
        pltpu.SemaphoreType.DMA,
    ],
)


def kernel(user_emb, item_emb, adj_indices, adj_values):
    ego0 = jnp.concatenate([user_emb, item_emb], axis=0)
    ego0p = jnp.concatenate([
        ego0[:HALF], jnp.zeros((HP - HALF, D), jnp.float32),
        ego0[HALF:], jnp.zeros((HP - HALF, D), jnp.float32),
    ], axis=0)
    e0_lo = ego0p[:, :DH]
    e0_hi = ego0p[:, DH:]
    src = adj_indices[0]
    dst = adj_indices[1]
    srcp = src + (HP - HALF) * (src >= HALF).astype(jnp.int32)
    dstp = dst + (HP - HALF) * (dst >= HALF).astype(jnp.int32)
    e = src.shape[0]
    pad = EPAD - e
    srcp = jnp.concatenate([srcp, jnp.zeros((pad,), jnp.int32)])
    dstp = jnp.concatenate([dstp, jnp.zeros((pad,), jnp.int32)])
    valp = jnp.concatenate([adj_values, jnp.zeros((pad,), jnp.float32)])

    e1_lo, e1_hi = _layer(e0_lo, e0_hi, srcp, dstp, valp)
    e2_lo, e2_hi = _layer(e1_lo, e1_hi, srcp, dstp, valp)
    fin_lo, fin_hi = _final(e2_lo, e2_hi, e0_lo, e0_hi, e1_lo, e1_hi,
                            srcp, dstp, valp)
    finalp = jnp.concatenate([fin_lo, fin_hi], axis=1)

    final_user = finalp[:NUM_USERS]
    final_item = jnp.concatenate(
        [finalp[NUM_USERS:HALF], finalp[HP:HP + NUM_ITEMS - (HALF - NUM_USERS)]],
        axis=0)
    return (final_user, final_item)


# speedup vs baseline: 5.0357x; 1.1173x over previous
import jax
import jax.numpy as jnp
from jax import lax
from jax.experimental import pallas as pl
from jax.experimental.pallas import tpu as pltpu
from jax.experimental.pallas import tpu_sc as plsc

NUM_USERS = 20000
NUM_ITEMS = 30000
D = 64
DH = 32
L = 16
NS = 16

HALF = 25000
HP = 25088
NPAD = 2 * HP
RPT = NPAD // NS

K = 128
G = 2
SK = G * K
NSC = 196
EDGES_PER_TILE = SK * NSC
EPAD = NS * EDGES_PER_TILE

MR = 64

_mesh = plsc.VectorSubcoreMesh(core_axis_name="c", subcore_axis_name="s")

_GDN = lax.GatherDimensionNumbers(
    offset_dims=(), collapsed_slice_dims=(0,), start_index_map=(0,))


def _lane_bcast(vec, lane):
    idx = jnp.full((L, 1), lane, jnp.int32)
    return lax.gather(vec, idx, dimension_numbers=_GDN, slice_sizes=(1,),
                      mode=lax.GatherScatterMode.PROMISE_IN_BOUNDS)


def _zero_acc(acc, rows2, s):
    def zero_row(r, _):
        rows2[0, r, pl.ds(0, L)] = jnp.zeros((L,), jnp.float32)
        rows2[0, r, pl.ds(L, L)] = jnp.zeros((L,), jnp.float32)
        return 0
    lax.fori_loop(0, SK, zero_row, 0)
    zbase = s * RPT
    for j in range(RPT // SK):
        pltpu.sync_copy(rows2.at[0], acc.at[pl.ds(zbase + j * SK, SK)])
    ztail = RPT - (RPT // SK) * SK
    pltpu.sync_copy(rows2.at[0, pl.ds(0, ztail)],
                    acc.at[pl.ds(zbase + (RPT // SK) * SK, ztail)])


def _edge_scan(ego, srcp, dstp, val, acc,
               rows2, srcp2, dstp2, val2, gsem0, gsem1, s):
    base = s * EDGES_PER_TILE
    gsems = (gsem0, gsem1)

    def load_idx(j, p):
        e0 = base + j * SK
        for g in range(G):
            pltpu.sync_copy(srcp.at[pl.ds(e0 + g * K, K)], srcp2.at[p, g])
            pltpu.sync_copy(dstp.at[pl.ds(e0 + g * K, K)], dstp2.at[p, g])
        pltpu.sync_copy(val.at[pl.ds(e0, SK)], val2.at[p])

    def issue_gathers(p):
        for g in range(G):
            pltpu.async_copy(ego.at[srcp2.at[p, g]],
                             rows2.at[p, pl.ds(g * K, K)], gsems[p])

    def wait_gathers(p):
        for g in range(G):
            pltpu.make_async_copy(ego.at[pl.ds(0, K)],
                                  rows2.at[p, pl.ds(g * K, K)],
                                  gsems[p]).wait()

    def scale(p):
        def sgroup(gr, _):
            vv = val2[p, pl.ds(gr * L, L)]
            for k16 in range(L):
                v = _lane_bcast(vv, k16)
                k = gr * L + k16
                rows2[p, k, pl.ds(0, L)] = rows2[p, k, pl.ds(0, L)] * v
                rows2[p, k, pl.ds(L, L)] = rows2[p, k, pl.ds(L, L)] * v
            return 0
        lax.fori_loop(0, SK // L, sgroup, 0)

    def scatter(p):
        for g in range(G):
            pltpu.sync_copy(rows2.at[p, pl.ds(g * K, K)],
                            acc.at[dstp2.at[p, g]], add=True)

    load_idx(0, 0)
    issue_gathers(0)

    def super_pair(jj, _):
        for p in (0, 1):
            j = 2 * jj + p
            jn = jnp.where(j + 1 >= NSC, 0, j + 1)
            load_idx(jn, 1 - p)
            issue_gathers(1 - p)
            wait_gathers(p)
            scale(p)
            scatter(p)
        return 0
    lax.fori_loop(0, NSC // 2, super_pair, 0)
    wait_gathers(0)


def _mean_out(e0, e1, e2, acc, out, b0, b1, b2, b3, s):
    r0 = s * RPT

    def mchunk(j, _):
        off = r0 + j * MR
        pltpu.sync_copy(e0.at[pl.ds(off, MR)], b0)
        pltpu.sync_copy(e1.at[pl.ds(off, MR)], b1)
        pltpu.sync_copy(e2.at[pl.ds(off, MR)], b2)
        pltpu.sync_copy(acc.at[pl.ds(off, MR)], b3)

        def srow(r, _):
            for cc in range(2):
                sl = pl.ds(cc * L, L)
                b0[r, sl] = ((b0[r, sl] + b1[r, sl]) +
                             (b2[r, sl] + b3[r, sl])) * 0.25
            return 0
        lax.fori_loop(0, MR, srow, 0)
        pltpu.sync_copy(b0, out.at[pl.ds(off, MR)])
        return 0

    lax.fori_loop(0, RPT // MR, mchunk, 0)


def _gcn_body(e0_lo, e0_hi, srcp, dstp, val,
              out_lo, out_hi, e1_lo, e1_hi, e2_lo, e2_hi,
              acc, rows2, srcp2, dstp2, val2, b0, b1, b2, b3, gsem0, gsem1):
    c = lax.axis_index("c")
    s = lax.axis_index("s")
    r0 = s * RPT

    def run(ego0, e1, e2, out):
        _zero_acc(acc, rows2, s)
        plsc.subcore_barrier()
        _edge_scan(ego0, srcp, dstp, val, acc,
                   rows2, srcp2, dstp2, val2, gsem0, gsem1, s)
        plsc.subcore_barrier()
        pltpu.sync_copy(acc.at[pl.ds(r0, RPT)], e1.at[pl.ds(r0, RPT)])
        _zero_acc(acc, rows2, s)
        plsc.subcore_barrier()
        _edge_scan(e1, srcp, dstp, val, acc,
                   rows2, srcp2, dstp2, val2, gsem0, gsem1, s)
        plsc.subcore_barrier()
        pltpu.sync_copy(acc.at[pl.ds(r0, RPT)], e2.at[pl.ds(r0, RPT)])
        _zero_acc(acc, rows2, s)
        plsc.subcore_barrier()
        _edge_scan(e2, srcp, dstp, val, acc,
                   rows2, srcp2, dstp2, val2, gsem0, gsem1, s)
        plsc.subcore_barrier()
        _mean_out(ego0, e1, e2, acc, out, b0, b1, b2, b3, s)

    @pl.when(c == 0)
    def _():
        run(e0_lo, e1_lo, e2_lo, out_lo)

    @pl.when(c == 1)
    def _():
        run(e0_hi, e1_hi, e2_hi, out_hi)


_half_t = jax.ShapeDtypeStruct((NPAD, DH), jnp.float32)

_gcn = pl.kernel(
    _gcn_body,
    mesh=_mesh,
    compiler_params=pltpu.CompilerParams(use_tc_tiling_on_sc=False),
    out_type=[_half_t, _half_t, _half_t, _half_t, _half_t, _half_t],
    scratch_types=[
        pltpu.VMEM_SHARED((NPAD, DH), jnp.float32),
        pltpu.VMEM((2, SK, DH), jnp.float32),
        pltpu.VMEM((2, G, K), jnp.int32),
        pltpu.VMEM((2, G, K), jnp.int32),
        pltpu.VMEM((2, SK), jnp.float32),
        pltpu.VMEM((MR, DH), jnp.float32),
        pltpu.VMEM((MR, DH), jnp.float32),
        pltpu.VMEM((MR, DH), jnp.float32),
        pltpu.VMEM((MR, DH), jnp.float32),
        pltpu.SemaphoreType.DMA,
        pltpu.SemaphoreType.DMA,
    ],
)


def kernel(user_emb, item_emb, adj_indices, adj_values):
    ego0 = jnp.concatenate([user_emb, item_emb], axis=0)
    ego0p = jnp.concatenate([
        ego0[:HALF], jnp.zeros((HP - HALF, D), jnp.float32),
        ego0[HALF:], jnp.zeros((HP - HALF, D), jnp.float32),
    ], axis=0)
    e0_lo = ego0p[:, :DH]
    e0_hi = ego0p[:, DH:]
    src = adj_indices[0]
    dst = adj_indices[1]
    srcp = src + (HP - HALF) * (src >= HALF).astype(jnp.int32)
    dstp = dst + (HP - HALF) * (dst >= HALF).astype(jnp.int32)
    e = src.shape[0]
    pad = EPAD - e
    srcp = jnp.concatenate([srcp, jnp.zeros((pad,), jnp.int32)])
    dstp = jnp.concatenate([dstp, jnp.zeros((pad,), jnp.int32)])
    valp = jnp.concatenate([adj_values, jnp.zeros((pad,), jnp.float32)])

    fin_lo, fin_hi, *_ = _gcn(e0_lo, e0_hi, srcp, dstp, valp)
    finalp = jnp.concatenate([fin_lo, fin_hi], axis=1)

    final_user = finalp[:NUM_USERS]
    final_item = jnp.concatenate(
        [finalp[NUM_USERS:HALF], finalp[HP:HP + NUM_ITEMS - (HALF - NUM_USERS)]],
        axis=0)
    return (final_user, final_item)
